# baseline (speedup 1.0000x reference)
import functools

import jax
import jax.numpy as jnp
from jax import lax
from jax.experimental import pallas as pl
from jax.experimental.pallas import tpu as pltpu
from jax.experimental.pallas import tpu_sc as plsc

NC = 2
NS = 16
NW = NC * NS
CH = 128
LANES = 16
NBUF = 5
TBUF = 4


def _make_transpose_body(V, D):
    n_full = V // (2 * D)
    n_even = (n_full // NW) * NW
    per_w = n_even // NW
    n_extra = n_full - n_even
    rem_cols = V - n_full * 2 * D
    assert per_w >= 2 * TBUF and (per_w - 2 * TBUF) % TBUF == 0
    n_m = 2 * D // LANES

    def body(tokT_hbm, tail2_hbm, tok2_hbm, in_v, out_v, isems, osems):
        wid = lax.axis_index("s") * NC + lax.axis_index("c")
        blk0 = wid * per_w
        iota = lax.iota(jnp.int32, LANES)
        row_idx = [lax.shift_right_logical(m * LANES + iota, 1) for m in range(n_m)]
        col_b = [((m * LANES + iota) & 1) * D for m in range(n_m)]

        def in_dma(bi, slot):
            return pltpu.make_async_copy(
                tokT_hbm.at[:, pl.ds(bi * 2 * D, 2 * D)], in_v.at[slot],
                isems.at[slot])

        def out_dma(bi, slot):
            return pltpu.make_async_copy(
                out_v.at[slot], tok2_hbm.at[pl.ds(bi * D, D)], osems.at[slot])

        def compute(slot):
            def cbody(c, c2):
                for m in range(n_m):
                    vec = in_v[slot, c, pl.ds(m * LANES, LANES)]
                    plsc.store_scatter(out_v.at[slot], [row_idx[m], col_b[m] + c],
                                       vec)
                return c2
            lax.fori_loop(0, D, cbody, 0)

        for s in range(TBUF):
            in_dma(blk0 + s, s).start()
        for j in range(TBUF):
            bi = blk0 + j
            in_dma(bi, j).wait()
            compute(j)
            out_dma(bi, j).start()
            in_dma(bi + TBUF, j).start()

        def main_body(j2, c2):
            for b in range(TBUF):
                j = TBUF + j2 * TBUF + b
                bi = blk0 + j
                in_dma(bi, b).wait()
                out_dma(bi - TBUF, b).wait()
                compute(b)
                out_dma(bi, b).start()
                in_dma(bi + TBUF, b).start()
            return c2

        lax.fori_loop(0, (per_w - 2 * TBUF) // TBUF, main_body, 0)

        for j in range(per_w - TBUF, per_w):
            bi = blk0 + j
            slot = j % TBUF
            in_dma(bi, slot).wait()
            out_dma(bi - TBUF, slot).wait()
            compute(slot)
            out_dma(bi, slot).start()
        for j in range(per_w - TBUF, per_w):
            out_dma(blk0 + j, j % TBUF).wait()

        @pl.when(wid < n_extra)
        def _():
            bi = n_even + wid
            in_dma(bi, 0).start()
            in_dma(bi, 0).wait()
            compute(0)
            out_dma(bi, 0).start()
            out_dma(bi, 0).wait()

        if rem_cols:
            n_tail = rem_cols // 2

            @pl.when(wid == n_extra)
            def _():
                pltpu.sync_copy(tail2_hbm, out_v.at[0, pl.ds(0, n_tail)])
                pltpu.sync_copy(out_v.at[0, pl.ds(0, n_tail)],
                                tok2_hbm.at[pl.ds(n_full * D, n_tail)])

    return body


def _make_gather_body(total, S, D):
    per_w = total // NW
    n_chunks = per_w // CH
    n_col = D // LANES
    assert n_chunks >= 2 * NBUF and (n_chunks - 2 * NBUF) % NBUF == 0

    def body(x_hbm, posdup_hbm, tok_hbm, out_hbm,
             pos_v, idx_v, rows_v, out_v, gsems, osems):
        wid = lax.axis_index("s") * NC + lax.axis_index("c")
        base0 = wid * per_w
        pltpu.sync_copy(posdup_hbm, pos_v)
        pltpu.sync_copy(x_hbm.at[pl.ds(base0, per_w)], idx_v)

        def gather(k, slot):
            return pltpu.make_async_copy(
                tok_hbm.at[idx_v.at[pl.ds(k * CH, CH)]], rows_v.at[slot],
                gsems.at[slot])

        def store(k, slot):
            base = base0 + k * CH
            return pltpu.make_async_copy(
                out_v.at[slot], out_hbm.at[pl.ds(base, CH)], osems.at[slot])

        def compute_chunk(k, slot):
            p0 = lax.rem(base0 + k * CH, S)

            def row_body(j, c2):
                r0 = j * LANES
                for i in range(LANES):
                    r = r0 + i
                    pr = p0 + r
                    for c in range(n_col):
                        sl = pl.ds(c * LANES, LANES)
                        out_v[slot, r, sl] = rows_v[slot, r, sl] + pos_v[pr, sl]
                return c2

            lax.fori_loop(0, CH // LANES, row_body, 0)

        for s in range(NBUF):
            gather(s, s).start()
        for k in range(NBUF):
            gather(k, k).wait()
            compute_chunk(k, k)
            store(k, k).start()
            gather(k + NBUF, k).start()

        def main_body(k2, c2):
            for b in range(NBUF):
                k = NBUF + k2 * NBUF + b
                gather(k, b).wait()
                store(k - NBUF, b).wait()
                compute_chunk(k, b)
                store(k, b).start()
                gather(k + NBUF, b).start()
            return c2

        lax.fori_loop(0, (n_chunks - 2 * NBUF) // NBUF, main_body, 0)

        for k in range(n_chunks - NBUF, n_chunks):
            slot = k % NBUF
            gather(k, slot).wait()
            store(k - NBUF, slot).wait()
            compute_chunk(k, slot)
            store(k, slot).start()
        for k in range(n_chunks - NBUF, n_chunks):
            store(k, k % NBUF).wait()

    return body


@functools.partial(jax.jit, static_argnames=())
def kernel(x, tok_table, pos_table):
    B, S = x.shape
    V, D = tok_table.shape
    total = B * S
    xf = x.reshape(total).astype(jnp.int32)
    posdup = jnp.concatenate([pos_table, pos_table], axis=0)

    mesh = plsc.VectorSubcoreMesh(core_axis_name="c", subcore_axis_name="s")

    transpose_run = pl.kernel(
        _make_transpose_body(V, D),
        mesh=mesh,
        compiler_params=pltpu.CompilerParams(needs_layout_passes=False),
        out_type=jax.ShapeDtypeStruct((V // 2, 2 * D), jnp.float32),
        scratch_types=[
            pltpu.VMEM((TBUF, D, 2 * D), jnp.float32),
            pltpu.VMEM((TBUF, D, 2 * D), jnp.float32),
            pltpu.SemaphoreType.DMA((TBUF,)),
            pltpu.SemaphoreType.DMA((TBUF,)),
        ],
    )
    n_full = V // (2 * D)
    tail2 = tok_table[n_full * 2 * D:].reshape(-1, 2 * D)
    tok2 = transpose_run(tok_table.T, tail2)
    tok_rm = tok2.reshape(V, D)

    gather_run = pl.kernel(
        _make_gather_body(total, S, D),
        mesh=mesh,
        compiler_params=pltpu.CompilerParams(use_tc_tiling_on_sc=False,
                                             needs_layout_passes=False),
        out_type=jax.ShapeDtypeStruct((total, D), jnp.float32),
        scratch_types=[
            pltpu.VMEM((2 * S, D), jnp.float32),
            pltpu.VMEM((total // NW,), jnp.int32),
            pltpu.VMEM((NBUF, CH, D), jnp.float32),
            pltpu.VMEM((NBUF, CH, D), jnp.float32),
            pltpu.SemaphoreType.DMA((NBUF,)),
            pltpu.SemaphoreType.DMA((NBUF,)),
        ],
    )
    out = gather_run(xf, posdup, tok_rm)
    return out.reshape(B, S, D)


# speedup vs baseline: 1.7103x; 1.7103x over previous
import functools

import jax
import jax.numpy as jnp
from jax import lax
from jax.experimental import pallas as pl
from jax.experimental.pallas import tpu as pltpu
from jax.experimental.pallas import tpu_sc as plsc

NC = 2
NS = 16
NW = NC * NS
CH = 128
LANES = 16
NBUF = 5


def _make_body(total, S, D):
    per_w = total // NW
    n_chunks = per_w // CH
    n_col = D // LANES
    assert n_chunks >= 2 * NBUF and (n_chunks - 2 * NBUF) % NBUF == 0

    def body(x_hbm, posdup_hbm, tok_hbm, out_hbm,
             pos_v, idx_v, rows_v, out_v, gsems, osems):
        wid = lax.axis_index("s") * NC + lax.axis_index("c")
        base0 = wid * per_w
        pltpu.sync_copy(posdup_hbm, pos_v)
        pltpu.sync_copy(x_hbm.at[pl.ds(base0, per_w)], idx_v)

        def gather(k, slot):
            return pltpu.make_async_copy(
                tok_hbm.at[idx_v.at[pl.ds(k * CH, CH)]], rows_v.at[slot],
                gsems.at[slot])

        def store(k, slot):
            base = base0 + k * CH
            return pltpu.make_async_copy(
                out_v.at[slot], out_hbm.at[pl.ds(base, CH)], osems.at[slot])

        def start_chunk(k, slot):
            gather(k, slot).start()

        def compute_chunk(k, slot):
            p0 = lax.rem(base0 + k * CH, S)

            def row_body(j, c2):
                r0 = j * LANES
                for i in range(LANES):
                    r = r0 + i
                    pr = p0 + r
                    for c in range(n_col):
                        sl = pl.ds(c * LANES, LANES)
                        out_v[slot, r, sl] = rows_v[slot, r, sl] + pos_v[pr, sl]
                return c2

            lax.fori_loop(0, CH // LANES, row_body, 0)

        for s in range(NBUF):
            start_chunk(s, s)
        for k in range(NBUF):
            gather(k, k).wait()
            compute_chunk(k, k)
            store(k, k).start()
            start_chunk(k + NBUF, k)

        def main_body(k2, c2):
            for b in range(NBUF):
                k = NBUF + k2 * NBUF + b
                gather(k, b).wait()
                store(k - NBUF, b).wait()
                compute_chunk(k, b)
                store(k, b).start()
                start_chunk(k + NBUF, b)
            return c2

        lax.fori_loop(0, (n_chunks - 2 * NBUF) // NBUF, main_body, 0)

        for k in range(n_chunks - NBUF, n_chunks):
            slot = k % NBUF
            gather(k, slot).wait()
            store(k - NBUF, slot).wait()
            compute_chunk(k, slot)
            store(k, slot).start()
        for k in range(n_chunks - NBUF, n_chunks):
            store(k, k % NBUF).wait()

    return body


@functools.partial(jax.jit, static_argnames=())
def kernel(x, tok_table, pos_table):
    B, S = x.shape
    V, D = tok_table.shape
    total = B * S
    xf = x.reshape(total).astype(jnp.int32)
    posdup = jnp.concatenate([pos_table, pos_table], axis=0)

    mesh = plsc.VectorSubcoreMesh(core_axis_name="c", subcore_axis_name="s")
    run = pl.kernel(
        _make_body(total, S, D),
        mesh=mesh,
        compiler_params=pltpu.CompilerParams(use_tc_tiling_on_sc=False),
        out_type=jax.ShapeDtypeStruct((total, D), jnp.float32),
        scratch_types=[
            pltpu.VMEM((2 * S, D), jnp.float32),
            pltpu.VMEM((total // NW,), jnp.int32),
            pltpu.VMEM((NBUF, CH, D), jnp.float32),
            pltpu.VMEM((NBUF, CH, D), jnp.float32),
            pltpu.SemaphoreType.DMA((NBUF,)),
            pltpu.SemaphoreType.DMA((NBUF,)),
        ],
    )
    out = run(xf, posdup, tok_table)
    return out.reshape(B, S, D)
